# baseline (speedup 1.0000x reference)
import functools
import math

import jax
import jax.numpy as jnp
from jax import lax
from jax.experimental import pallas as pl
from jax.experimental.pallas import tpu as pltpu
from jax.experimental.pallas import tpu_sc as plsc

_NUM_CORES = 2
_NUM_SUBCORES = 16
_NUM_WORKERS = _NUM_CORES * _NUM_SUBCORES
_LANES = 16

_D = 64
_SCALE = math.sqrt(_D)
_GROUP = 128
_CHUNK = 512
_GPC = _CHUNK // _GROUP


@functools.partial(jax.jit, static_argnames=("batch",))
def _embed(lut, idx2d, batch):
    n_rows = batch
    rows_per_w = n_rows // _NUM_WORKERS
    chunks_per_w = rows_per_w // _CHUNK
    groups_per_w = rows_per_w // _GROUP

    mesh = plsc.VectorSubcoreMesh(
        core_axis_name="c", subcore_axis_name="s",
        num_cores=_NUM_CORES, num_subcores=_NUM_SUBCORES,
    )

    @functools.partial(
        pl.kernel,
        mesh=mesh,
        out_type=jax.ShapeDtypeStruct((n_rows, _D), jnp.float32),
        scratch_types=[
            pltpu.VMEM((_GPC, _GROUP), jnp.int32),
            pltpu.VMEM((_CHUNK, _D), jnp.float32),
            pltpu.SemaphoreType.DMA,
        ],
        compiler_params=pltpu.CompilerParams(use_tc_tiling_on_sc=False),
    )
    def k(table_hbm, idx_hbm, out_hbm, idx_v, rows_v, sem):
        wid = lax.axis_index("s") * _NUM_CORES + lax.axis_index("c")
        gbase = wid * groups_per_w
        rbase = wid * rows_per_w

        def chunk_body(i, carry):
            pltpu.sync_copy(idx_hbm.at[pl.ds(gbase + i * _GPC, _GPC)], idx_v)
            cps = [
                pltpu.async_copy(
                    table_hbm.at[idx_v.at[j]],
                    rows_v.at[pl.ds(j * _GROUP, _GROUP)],
                    sem,
                )
                for j in range(_GPC)
            ]
            for cp in cps:
                cp.wait()

            def scale_body(r, c2):
                for c in range(_D // _LANES):
                    sl = pl.ds(c * _LANES, _LANES)
                    rows_v[r, sl] = rows_v[r, sl] * _SCALE
                return c2

            lax.fori_loop(0, _CHUNK, scale_body, 0, unroll=2)
            pltpu.sync_copy(rows_v, out_hbm.at[pl.ds(rbase + i * _CHUNK, _CHUNK)])
            return carry

        lax.fori_loop(0, chunks_per_w, chunk_body, 0)

    return k(lut, idx2d)


def kernel(x, lut):
    b, h = x.shape
    batch = b * h
    idx2d = x.reshape(batch // _GROUP, _GROUP)
    out = _embed(lut, idx2d, batch)
    return out.reshape(b, h, _D)


# speedup vs baseline: 1.0699x; 1.0699x over previous
import functools
import math

import jax
import jax.numpy as jnp
from jax import lax
from jax.experimental import pallas as pl
from jax.experimental.pallas import tpu as pltpu
from jax.experimental.pallas import tpu_sc as plsc

_NUM_CORES = 2
_NUM_SUBCORES = 16
_NUM_WORKERS = _NUM_CORES * _NUM_SUBCORES
_LANES = 16

_D = 64
_SCALE = math.sqrt(_D)
_GROUP = 128
_CHUNK = 512
_GPC = _CHUNK // _GROUP


@functools.partial(jax.jit, static_argnames=("batch",))
def _embed(lut, idx2d, batch):
    n_rows = batch
    rows_per_w = n_rows // _NUM_WORKERS
    chunks_per_w = rows_per_w // _CHUNK
    groups_per_w = rows_per_w // _GROUP

    mesh = plsc.VectorSubcoreMesh(
        core_axis_name="c", subcore_axis_name="s",
        num_cores=_NUM_CORES, num_subcores=_NUM_SUBCORES,
    )

    @functools.partial(
        pl.kernel,
        mesh=mesh,
        out_type=jax.ShapeDtypeStruct((n_rows, _D), jnp.float32),
        scratch_types=[
            pltpu.VMEM((2, _GPC, _GROUP), jnp.int32),
            pltpu.VMEM((2, _CHUNK, _D), jnp.float32),
            pltpu.SemaphoreType.DMA,
            pltpu.SemaphoreType.DMA,
        ],
        compiler_params=pltpu.CompilerParams(use_tc_tiling_on_sc=False),
    )
    def k(table_hbm, idx_hbm, out_hbm, idx_v, rows_v, sem_g, sem_o):
        wid = lax.axis_index("s") * _NUM_CORES + lax.axis_index("c")
        gbase = wid * groups_per_w
        rbase = wid * rows_per_w
        n_chunks = chunks_per_w

        def issue_gathers(i, b):
            pltpu.sync_copy(idx_hbm.at[pl.ds(gbase + i * _GPC, _GPC)],
                            idx_v.at[b])
            for j in range(_GPC):
                pltpu.async_copy(
                    table_hbm.at[idx_v.at[b, j]],
                    rows_v.at[b, pl.ds(j * _GROUP, _GROUP)],
                    sem_g,
                )

        def drain_gathers(b):
            pltpu.make_async_copy(out_hbm.at[pl.ds(0, _CHUNK)],
                                  rows_v.at[b], sem_g).wait()

        def drain_out(b):
            pltpu.make_async_copy(rows_v.at[b],
                                  out_hbm.at[pl.ds(0, _CHUNK)], sem_o).wait()

        def scale_chunk(b):
            def scale_body(r, c2):
                for c in range(_D // _LANES):
                    sl = pl.ds(c * _LANES, _LANES)
                    rows_v[b, r, sl] = rows_v[b, r, sl] * _SCALE
                return c2

            lax.fori_loop(0, _CHUNK, scale_body, 0, unroll=4)

        issue_gathers(0, 0)

        def pair_body(ii, carry):
            for b in range(2):
                i = 2 * ii + b
                q = 1 - b
                drain_gathers(b)

                @pl.when(i + 1 < n_chunks)
                def _():
                    @pl.when(i > 0)
                    def _():
                        drain_out(q)
                    issue_gathers(i + 1, q)

                scale_chunk(b)
                pltpu.async_copy(rows_v.at[b],
                                 out_hbm.at[pl.ds(rbase + i * _CHUNK, _CHUNK)],
                                 sem_o)
            return carry

        lax.fori_loop(0, n_chunks // 2, pair_body, 0)
        drain_out(0)
        drain_out(1)

    return k(lut, idx2d)


def kernel(x, lut):
    b, h = x.shape
    batch = b * h
    idx2d = x.reshape(batch // _GROUP, _GROUP)
    out = _embed(lut, idx2d, batch)
    return out.reshape(b, h, _D)


# speedup vs baseline: 1.0725x; 1.0024x over previous
import functools
import math

import jax
import jax.numpy as jnp
from jax import lax
from jax.experimental import pallas as pl
from jax.experimental.pallas import tpu as pltpu
from jax.experimental.pallas import tpu_sc as plsc

_NUM_CORES = 2
_NUM_SUBCORES = 16
_NUM_WORKERS = _NUM_CORES * _NUM_SUBCORES
_LANES = 16

_D = 64
_SCALE = math.sqrt(_D)
_GROUP = 128
_CHUNK = 512
_GPC = _CHUNK // _GROUP


@functools.partial(jax.jit, static_argnames=("batch",))
def _embed(lut, idx2d, batch):
    n_rows = batch
    rows_per_w = n_rows // _NUM_WORKERS
    chunks_per_w = rows_per_w // _CHUNK
    groups_per_w = rows_per_w // _GROUP

    mesh = plsc.VectorSubcoreMesh(
        core_axis_name="c", subcore_axis_name="s",
        num_cores=_NUM_CORES, num_subcores=_NUM_SUBCORES,
    )

    @functools.partial(
        pl.kernel,
        mesh=mesh,
        out_type=jax.ShapeDtypeStruct((n_rows, _D), jnp.float32),
        scratch_types=[
            pltpu.VMEM((2, _GPC, _GROUP), jnp.int32),
            pltpu.VMEM((2, _CHUNK, _D), jnp.float32),
            pltpu.SemaphoreType.DMA,
            pltpu.SemaphoreType.DMA,
        ],
        compiler_params=pltpu.CompilerParams(use_tc_tiling_on_sc=False),
    )
    def k(table_hbm, idx_hbm, out_hbm, idx_v, rows_v, sem_g, sem_o):
        wid = lax.axis_index("s") * _NUM_CORES + lax.axis_index("c")
        gbase = wid * groups_per_w
        rbase = wid * rows_per_w
        n_chunks = chunks_per_w

        def issue_gathers(i, b):
            pltpu.sync_copy(idx_hbm.at[pl.ds(gbase + i * _GPC, _GPC)],
                            idx_v.at[b])
            for j in range(_GPC):
                pltpu.async_copy(
                    table_hbm.at[idx_v.at[b, j]],
                    rows_v.at[b, pl.ds(j * _GROUP, _GROUP)],
                    sem_g,
                )

        def drain_gathers(b):
            pltpu.make_async_copy(out_hbm.at[pl.ds(0, _CHUNK)],
                                  rows_v.at[b], sem_g).wait()

        def drain_out(b):
            pltpu.make_async_copy(rows_v.at[b],
                                  out_hbm.at[pl.ds(0, _CHUNK)], sem_o).wait()

        def scale_chunk(b):
            @plsc.parallel_loop(0, _CHUNK, unroll=4)
            def _(r):
                for c in range(_D // _LANES):
                    sl = pl.ds(c * _LANES, _LANES)
                    rows_v[b, r, sl] = rows_v[b, r, sl] * _SCALE

        issue_gathers(0, 0)

        def pair_body(ii, carry):
            for b in range(2):
                i = 2 * ii + b
                q = 1 - b
                drain_gathers(b)

                @pl.when(i + 1 < n_chunks)
                def _():
                    @pl.when(i > 0)
                    def _():
                        drain_out(q)
                    issue_gathers(i + 1, q)

                scale_chunk(b)
                pltpu.async_copy(rows_v.at[b],
                                 out_hbm.at[pl.ds(rbase + i * _CHUNK, _CHUNK)],
                                 sem_o)
            return carry

        lax.fori_loop(0, n_chunks // 2, pair_body, 0)
        drain_out(0)
        drain_out(1)

    return k(lut, idx2d)


def kernel(x, lut):
    b, h = x.shape
    batch = b * h
    idx2d = x.reshape(batch // _GROUP, _GROUP)
    out = _embed(lut, idx2d, batch)
    return out.reshape(b, h, _D)
